# Initial kernel scaffold; baseline (speedup 1.0000x reference)
import jax
import jax.numpy as jnp
from jax.experimental import pallas as pl


def kernel(X_o, edge_index, edge_weight, W1, W2):
    raise NotImplementedError("write your pallas kernel here")


# speedup vs baseline: 5.3788x; 5.3788x over previous
import functools

import jax
import jax.numpy as jnp
from jax import lax
from jax.experimental import pallas as pl
from jax.experimental.pallas import tpu as pltpu
from jax.experimental.pallas import tpu_sc as plsc

_N = 10000
_E = 320000
_CH = 80
_G = _CH // 16
_NW = 32
_EPW = _E // _NW
_NCH = _EPW // _CH
_RPT = 624
_REM = _N - 16 * _RPT
_ZR = 208


def _make_edge_kernel(D):
  mesh = plsc.VectorSubcoreMesh(core_axis_name="c", subcore_axis_name="s")

  @functools.partial(
      pl.kernel,
      mesh=mesh,
      out_type=jax.ShapeDtypeStruct((2, _N, D), jnp.float32),
      scratch_types=[
          pltpu.VMEM((_EPW,), jnp.int32),
          pltpu.VMEM((_EPW,), jnp.int32),
          pltpu.VMEM((_EPW,), jnp.float32),
          pltpu.VMEM((_CH,), jnp.int32),
          pltpu.VMEM((_CH,), jnp.int32),
          pltpu.VMEM((_CH, D), jnp.float32),
          pltpu.VMEM_SHARED((_N, D), jnp.float32),
          pltpu.SemaphoreType.DMA,
      ],
  )
  def ek(h_hbm, src_hbm, dst_hbm, w_hbm, out_hbm,
         src_v, dst_v, w_v, sidx_v, didx_v, rows_v, acc, sem):
    c = lax.axis_index("c")
    s = lax.axis_index("s")
    wid = s * 2 + c

    zv = jnp.zeros((16,), jnp.float32)

    def zrow(r, carry):
      for j in range(D // 16):
        rows_v[r, pl.ds(j * 16, 16)] = zv
      return carry

    lax.fori_loop(0, _CH, zrow, 0)
    for t in range(_RPT // _CH):
      pltpu.sync_copy(rows_v, acc.at[pl.ds(s * _RPT + t * _CH, _CH)])
    pltpu.sync_copy(rows_v.at[pl.ds(0, _RPT % _CH)],
                    acc.at[pl.ds(s * _RPT + (_RPT // _CH) * _CH, _RPT % _CH)])

    @pl.when(s == 0)
    def _():
      pltpu.sync_copy(rows_v.at[pl.ds(0, _REM)], acc.at[pl.ds(16 * _RPT, _REM)])

    plsc.subcore_barrier()

    e0 = wid * _EPW
    pltpu.sync_copy(src_hbm.at[pl.ds(e0, _EPW)], src_v)
    pltpu.sync_copy(dst_hbm.at[pl.ds(e0, _EPW)], dst_v)
    pltpu.sync_copy(w_hbm.at[pl.ds(e0, _EPW)], w_v)

    def chunk(i, carry):
      base = i * _CH
      for g in range(_G):
        pos = pl.ds(g * 16, 16)
        sidx_v[pos] = src_v[pl.ds(base + g * 16, 16)]
        didx_v[pos] = dst_v[pl.ds(base + g * 16, 16)]

      pltpu.async_copy(h_hbm.at[sidx_v], rows_v, sem).wait()

      for g in range(_G):
        wv16 = w_v[pl.ds(base + g * 16, 16)]
        for l in range(16):
          wl = jnp.broadcast_to(wv16[l], (16,))
          r = g * 16 + l
          for j in range(D // 16):
            sl = pl.ds(j * 16, 16)
            rows_v[r, sl] = rows_v[r, sl] * wl

      pltpu.sync_copy(rows_v, acc.at[didx_v], add=True)
      return carry

    lax.fori_loop(0, _NCH, chunk, 0)

    plsc.subcore_barrier()
    pltpu.sync_copy(acc.at[pl.ds(s * _RPT, _RPT)],
                    out_hbm.at[c, pl.ds(s * _RPT, _RPT)])

    @pl.when(s == 0)
    def _():
      pltpu.sync_copy(acc.at[pl.ds(16 * _RPT, _REM)],
                      out_hbm.at[c, pl.ds(16 * _RPT, _REM)])

  return ek


_edge128 = _make_edge_kernel(128)

_BR = 400


def _elu(x):
  return jnp.where(x > 0, x, jnp.exp(x) - 1.0)


def _mm(x, w):
  n, k = x.shape
  m = w.shape[1]

  def body(x_ref, w_ref, o_ref):
    o_ref[...] = jnp.dot(x_ref[...], w_ref[...],
                         preferred_element_type=jnp.float32)

  return pl.pallas_call(
      body,
      grid=(n // _BR,),
      in_specs=[pl.BlockSpec((_BR, k), lambda i: (i, 0)),
                pl.BlockSpec((k, m), lambda i: (0, 0))],
      out_specs=pl.BlockSpec((_BR, m), lambda i: (i, 0)),
      out_shape=jax.ShapeDtypeStruct((n, m), jnp.float32),
  )(x, w)


def _combine_mm(p, w):
  _, n, k = p.shape
  m = w.shape[1]

  def body(p_ref, w_ref, o_ref):
    z = _elu(p_ref[0] + p_ref[1])
    o_ref[...] = jnp.dot(z, w_ref[...], preferred_element_type=jnp.float32)

  return pl.pallas_call(
      body,
      grid=(n // _BR,),
      in_specs=[pl.BlockSpec((2, _BR, k), lambda i: (0, i, 0)),
                pl.BlockSpec((k, m), lambda i: (0, 0))],
      out_specs=pl.BlockSpec((_BR, m), lambda i: (i, 0)),
      out_shape=jax.ShapeDtypeStruct((n, m), jnp.float32),
  )(p, w)


def _combine_elu(p):
  _, n, k = p.shape

  def body(p_ref, o_ref):
    o_ref[...] = _elu(p_ref[0] + p_ref[1])

  return pl.pallas_call(
      body,
      grid=(n // _BR,),
      in_specs=[pl.BlockSpec((2, _BR, k), lambda i: (0, i, 0))],
      out_specs=pl.BlockSpec((_BR, k), lambda i: (i, 0)),
      out_shape=jax.ShapeDtypeStruct((n, k), jnp.float32),
  )(p)


def kernel(X_o, edge_index, edge_weight, W1, W2):
  src_r = edge_index[0]
  dst_r = edge_index[1]
  w_r = edge_weight

  h = _mm(X_o, W1)
  p1 = _edge128(h, src_r, dst_r, w_r)
  h2 = _combine_mm(p1, W2)
  h2p = jnp.pad(h2, ((0, 0), (0, 64)))
  p2 = _edge128(h2p, src_r, dst_r, w_r)
  return _combine_elu(p2[:, :, :64])


# speedup vs baseline: 6.8651x; 1.2763x over previous
import functools

import jax
import jax.numpy as jnp
from jax import lax
from jax.experimental import pallas as pl
from jax.experimental.pallas import tpu as pltpu
from jax.experimental.pallas import tpu_sc as plsc

_N = 10000
_E = 320000
_CH = 80
_G = _CH // 16
_NW = 32
_EPW = _E // _NW
_NCH = _EPW // _CH
_NB = 3
_RPT = 624
_REM = _N - 16 * _RPT


def _make_edge_kernel(D, tc_tiling=None):
  mesh = plsc.VectorSubcoreMesh(core_axis_name="c", subcore_axis_name="s")

  @functools.partial(
      pl.kernel,
      mesh=mesh,
      compiler_params=pltpu.CompilerParams(use_tc_tiling_on_sc=tc_tiling),
      out_type=jax.ShapeDtypeStruct((2, _N, D), jnp.float32),
      scratch_types=(
          [pltpu.VMEM((_EPW,), jnp.float32)]
          + [pltpu.VMEM((_CH,), jnp.int32)] * _NB
          + [pltpu.VMEM((_CH,), jnp.int32)] * _NB
          + [pltpu.VMEM((_CH, D), jnp.float32)] * _NB
          + [pltpu.VMEM_SHARED((_N, D), jnp.float32)]
          + [pltpu.SemaphoreType.DMA] * (3 * _NB)
      ),
  )
  def ek(h_hbm, src_hbm, dst_hbm, w_hbm, out_hbm, w_v, *bufs):
    sidx = bufs[0:_NB]
    didx = bufs[_NB:2 * _NB]
    rows = bufs[2 * _NB:3 * _NB]
    acc = bufs[3 * _NB]
    isem = bufs[3 * _NB + 1:3 * _NB + 1 + _NB]
    gsem = bufs[3 * _NB + 1 + _NB:3 * _NB + 1 + 2 * _NB]
    ssem = bufs[3 * _NB + 1 + 2 * _NB:]

    c = lax.axis_index("c")
    s = lax.axis_index("s")
    wid = s * 2 + c
    e0 = wid * _EPW

    zv = jnp.zeros((16,), jnp.float32)

    def zrow(r, carry):
      for j in range(D // 16):
        rows[0][r, pl.ds(j * 16, 16)] = zv
      return carry

    lax.fori_loop(0, _CH, zrow, 0)
    for t in range(_RPT // _CH):
      pltpu.sync_copy(rows[0], acc.at[pl.ds(s * _RPT + t * _CH, _CH)])
    pltpu.sync_copy(rows[0].at[pl.ds(0, _RPT % _CH)],
                    acc.at[pl.ds(s * _RPT + (_RPT // _CH) * _CH, _RPT % _CH)])

    @pl.when(s == 0)
    def _():
      pltpu.sync_copy(rows[0].at[pl.ds(0, _REM)], acc.at[pl.ds(16 * _RPT, _REM)])

    pltpu.sync_copy(w_hbm.at[pl.ds(e0, _EPW)], w_v)
    plsc.subcore_barrier()

    def idx_start(ci, b):
      off = e0 + ci * _CH
      pltpu.async_copy(src_hbm.at[pl.ds(off, _CH)], sidx[b], isem[b])
      pltpu.async_copy(dst_hbm.at[pl.ds(off, _CH)], didx[b], isem[b])

    def idx_wait(ci, b):
      off = e0 + ci * _CH
      pltpu.make_async_copy(src_hbm.at[pl.ds(off, _CH)], sidx[b], isem[b]).wait()
      pltpu.make_async_copy(dst_hbm.at[pl.ds(off, _CH)], didx[b], isem[b]).wait()

    def gather_start(b):
      pltpu.async_copy(h_hbm.at[sidx[b]], rows[b], gsem[b])

    def gather_wait(b):
      pltpu.make_async_copy(h_hbm.at[sidx[b]], rows[b], gsem[b]).wait()

    def scat_start(b):
      pltpu.async_copy(rows[b], acc.at[didx[b]], ssem[b], add=True)

    def scat_wait(b):
      pltpu.make_async_copy(rows[b], acc.at[didx[b]], ssem[b]).wait()

    def scale(ci, b):
      rv = rows[b]
      for g in range(_G):
        wv16 = w_v[pl.ds(ci * _CH + g * 16, 16)]
        for l in range(16):
          wl = jnp.broadcast_to(wv16[l], (16,))
          r = g * 16 + l
          for j in range(D // 16):
            sl = pl.ds(j * 16, 16)
            rv[r, sl] = rv[r, sl] * wl

    for b in range(_NB - 1):
      idx_start(b, b)
      idx_wait(b, b)
      gather_start(b)

    def triple(ip, carry):
      for b in range(_NB):
        ci = _NB * ip + b
        nb = (b + _NB - 1) % _NB

        @pl.when(ci > 0)
        def _():
          scat_wait(nb)

        idx_start(ci + _NB - 1, nb)
        gather_wait(b)
        idx_wait(ci + _NB - 1, nb)
        gather_start(nb)
        scale(ci, b)
        scat_start(b)
      return carry

    ntrip = (_NCH - (_NB - 1)) // _NB
    lax.fori_loop(0, ntrip, triple, 0)

    scat_wait(2)
    gather_wait(0)
    scale(_NCH - 2, 0)
    scat_start(0)
    gather_wait(1)
    scale(_NCH - 1, 1)
    scat_start(1)
    scat_wait(0)
    scat_wait(1)

    plsc.subcore_barrier()
    pltpu.sync_copy(acc.at[pl.ds(s * _RPT, _RPT)],
                    out_hbm.at[c, pl.ds(s * _RPT, _RPT)])

    @pl.when(s == 0)
    def _():
      pltpu.sync_copy(acc.at[pl.ds(16 * _RPT, _REM)],
                      out_hbm.at[c, pl.ds(16 * _RPT, _REM)])

  return ek


_edge128 = _make_edge_kernel(128)
_edge64 = _make_edge_kernel(64, tc_tiling=False)

_BR = 400


def _elu(x):
  return jnp.where(x > 0, x, jnp.exp(x) - 1.0)


def _mm(x, w):
  n, k = x.shape
  m = w.shape[1]

  def body(x_ref, w_ref, o_ref):
    o_ref[...] = jnp.dot(x_ref[...], w_ref[...],
                         preferred_element_type=jnp.float32)

  return pl.pallas_call(
      body,
      grid=(n // _BR,),
      in_specs=[pl.BlockSpec((_BR, k), lambda i: (i, 0)),
                pl.BlockSpec((k, m), lambda i: (0, 0))],
      out_specs=pl.BlockSpec((_BR, m), lambda i: (i, 0)),
      out_shape=jax.ShapeDtypeStruct((n, m), jnp.float32),
  )(x, w)


def _combine_mm(p, w):
  _, n, k = p.shape
  m = w.shape[1]

  def body(p_ref, w_ref, o_ref):
    z = _elu(p_ref[0] + p_ref[1])
    o_ref[...] = jnp.dot(z, w_ref[...], preferred_element_type=jnp.float32)

  return pl.pallas_call(
      body,
      grid=(n // _BR,),
      in_specs=[pl.BlockSpec((2, _BR, k), lambda i: (0, i, 0)),
                pl.BlockSpec((k, m), lambda i: (0, 0))],
      out_specs=pl.BlockSpec((_BR, m), lambda i: (i, 0)),
      out_shape=jax.ShapeDtypeStruct((n, m), jnp.float32),
  )(p, w)


def _combine_elu(p):
  _, n, k = p.shape

  def body(p_ref, o_ref):
    o_ref[...] = _elu(p_ref[0] + p_ref[1])

  return pl.pallas_call(
      body,
      grid=(n // _BR,),
      in_specs=[pl.BlockSpec((2, _BR, k), lambda i: (0, i, 0))],
      out_specs=pl.BlockSpec((_BR, k), lambda i: (i, 0)),
      out_shape=jax.ShapeDtypeStruct((n, k), jnp.float32),
  )(p)


def kernel(X_o, edge_index, edge_weight, W1, W2):
  src_r = edge_index[0]
  dst_r = edge_index[1]
  w_r = edge_weight

  h = _mm(X_o, W1)
  p1 = _edge128(h, src_r, dst_r, w_r)
  h2 = _combine_mm(p1, W2)
  p2 = _edge64(h2, src_r, dst_r, w_r)
  return _combine_elu(p2)
